# baseline (speedup 1.0000x reference)
import math

import jax
import jax.numpy as jnp
from jax import lax
from jax.experimental import pallas as pl
from jax.experimental.pallas import tpu as pltpu
from jax.experimental.pallas import tpu_sc as plsc

N = 2048
C = 1024
E = 8

_F32 = jnp.float32
_LOG8_7 = math.log(8.0) / 7.0

NW = 32
TPW = N // NW
NG = TPW // 16


def _route_kernel(x_ref, de_ref, wc_ref, bc_ref, g_ref, bt_ref,
                  wg_ref, bg_ref, logits_ref):
    x = x_ref[...]
    conv = lax.dot_general(
        x, wc_ref[...], (((1,), (1,)), ((), ())),
        preferred_element_type=_F32) + bc_ref[...]
    mu = jnp.mean(conv, axis=1, keepdims=True)
    var = jnp.mean((conv - mu) ** 2, axis=1, keepdims=True)
    route = ((conv - mu) * lax.rsqrt(var + 1e-5) * g_ref[...]
             + bt_ref[...] + de_ref[...])
    logits_ref[...] = lax.dot_general(
        wg_ref[...], route, (((1,), (1,)), ((), ())),
        preferred_element_type=_F32) + bg_ref[...]


def _sc_route_body(logits_hbm, wte_hbm, sums_hbm, lv, wv, sv):
    wid = lax.axis_index("s") * 2 + lax.axis_index("c")
    pltpu.sync_copy(logits_hbm.at[wid], lv)
    msums = [jnp.zeros((16,), _F32) for _ in range(E)]
    wsums = [jnp.zeros((16,), _F32) for _ in range(E)]
    for g in range(NG):
        sl = pl.ds(g * 16, 16)
        l = [lv[e, sl] for e in range(E)]
        m = l[0]
        for e in range(1, E):
            m = jnp.maximum(m, l[e])
        ex = [jnp.exp(l[e] - m) for e in range(E)]
        z = ex[0]
        for e in range(1, E):
            z = z + ex[e]
        w = [ex[e] / z for e in range(E)]
        s = w[0] * l[0]
        for e in range(1, E):
            s = s + w[e] * l[e]
        rank = []
        for e in range(E):
            r = jnp.zeros((16,), _F32)
            for j in range(E):
                if j == e:
                    continue
                beats = w[j] > w[e]
                if j < e:
                    beats = beats | (w[j] == w[e])
                r = r + jnp.where(beats, 1.0, 0.0)
            rank.append(r)
        for e in range(E):
            thresh = jnp.exp(s - m + (rank[e] - 1.0) * _LOG8_7)
            sel = (rank[e] == 0.0) | (z > thresh)
            wv[e, sl] = jnp.where(sel, w[e], 0.0)
            msums[e] = msums[e] + jnp.where(sel, 1.0, 0.0)
            wsums[e] = wsums[e] + w[e]
    for e in range(E):
        sv[e, :] = msums[e]
        sv[E + e, :] = wsums[e]
    pltpu.sync_copy(wv, wte_hbm.at[wid])
    pltpu.sync_copy(sv, sums_hbm.at[wid])


def _share_kernel(x_ref, ws_ref, share_ref):
    share_ref[...] = lax.dot_general(
        x_ref[...], ws_ref[...], (((1,), (1,)), ((), ())),
        preferred_element_type=_F32)


def _mix_kernel(x_ref, wte_t_ref, sums_ref, share_ref, we_ref, ba_ref,
                y_ref, loss_ref, wte_ref):
    s = pl.program_id(1)

    @pl.when(s == 0)
    def _init():
        ones = jnp.ones((1, N), _F32)
        wte_ref[...] = jnp.concatenate([wte_t_ref[...], ones], axis=0).T
        y_ref[...] = share_ref[...] + lax.dot_general(
            wte_ref[...], ba_ref[...], (((1,), (0,)), ((), ())),
            preferred_element_type=_F32)

    @pl.when(s > 0)
    def _expert():
        e = s - 1
        xw = lax.dot_general(
            x_ref[...], we_ref[0], (((1,), (1,)), ((), ())),
            preferred_element_type=_F32)
        onehot = (lax.broadcasted_iota(jnp.int32, (E + 1, 1), 0)
                  == e).astype(_F32)
        wcol = lax.dot_general(
            wte_ref[...], onehot, (((1,), (0,)), ((), ())),
            preferred_element_type=_F32)
        y_ref[...] += wcol * xw

    @pl.when(s == E)
    def _loss():
        sm = sums_ref[...]
        t0 = jnp.sum(sm, axis=0)
        tot = jnp.sum(t0, axis=1, keepdims=True)
        prod = tot[:E, :] * tot[E:, :] * (1.0 / (N * N))
        loss_ref[...] = jnp.sum(prod, axis=0, keepdims=True) * (
            float(E * E) / float(E))


@jax.jit
def _moe(features, domain_emb, W_share, b_share, W_conv, b_conv,
         ln_gamma, ln_beta, W_gate, b_gate, W_experts, b_experts):
    de = domain_emb.reshape(1, C)
    bc = b_conv.reshape(1, C)
    g = ln_gamma.reshape(1, C)
    bt = ln_beta.reshape(1, C)
    bg = b_gate.reshape(E, 1)
    b_all = jnp.concatenate([b_experts, b_share[None]], axis=0)

    full2 = lambda *_: (0, 0)

    logits_t = pl.pallas_call(
        _route_kernel,
        grid=(1,),
        in_specs=[
            pl.BlockSpec((N, C), lambda i: (0, 0)),
            pl.BlockSpec((1, C), lambda i: (0, 0)),
            pl.BlockSpec((C, C), lambda i: (0, 0)),
            pl.BlockSpec((1, C), lambda i: (0, 0)),
            pl.BlockSpec((1, C), lambda i: (0, 0)),
            pl.BlockSpec((1, C), lambda i: (0, 0)),
            pl.BlockSpec((E, C), lambda i: (0, 0)),
            pl.BlockSpec((E, 1), lambda i: (0, 0)),
        ],
        out_specs=pl.BlockSpec((E, N), lambda i: (0, 0)),
        out_shape=jax.ShapeDtypeStruct((E, N), _F32),
    )(features, de, W_conv, bc, g, bt, W_gate, bg)

    logits3 = logits_t.reshape(E, NW, TPW).transpose(1, 0, 2)
    mesh = plsc.VectorSubcoreMesh(core_axis_name="c", subcore_axis_name="s")
    wte3, sums3 = pl.kernel(
        _sc_route_body,
        mesh=mesh,
        out_type=[
            jax.ShapeDtypeStruct((NW, E, TPW), _F32),
            jax.ShapeDtypeStruct((NW, 2 * E, 16), _F32),
        ],
        scratch_types=[
            pltpu.VMEM((E, TPW), _F32),
            pltpu.VMEM((E, TPW), _F32),
            pltpu.VMEM((2 * E, 16), _F32),
        ],
    )(logits3)
    wte_t = wte3.transpose(1, 0, 2).reshape(E, N)

    share = pl.pallas_call(
        _share_kernel,
        grid=(1,),
        in_specs=[
            pl.BlockSpec((N, C), lambda i: (0, 0)),
            pl.BlockSpec((C, C), lambda i: (0, 0)),
        ],
        out_specs=pl.BlockSpec((N, C), lambda i: (0, 0)),
        out_shape=jax.ShapeDtypeStruct((N, C), _F32),
    )(features, W_share)

    y, loss = pl.pallas_call(
        _mix_kernel,
        grid=(1, E + 1),
        in_specs=[
            pl.BlockSpec((N, C), full2),
            pl.BlockSpec((E, N), full2),
            pl.BlockSpec((NW, 2 * E, 16), lambda i, s: (0, 0, 0)),
            pl.BlockSpec((N, C), full2),
            pl.BlockSpec((1, C, C),
                         lambda i, s: (jnp.maximum(s - 1, 0), 0, 0)),
            pl.BlockSpec((E + 1, C), full2),
        ],
        out_specs=[
            pl.BlockSpec((N, C), full2),
            pl.BlockSpec((1, 1), full2),
        ],
        out_shape=[
            jax.ShapeDtypeStruct((N, C), _F32),
            jax.ShapeDtypeStruct((1, 1), _F32),
        ],
        scratch_shapes=[
            pltpu.VMEM((N, E + 1), _F32),
        ],
        compiler_params=pltpu.CompilerParams(
            dimension_semantics=("arbitrary", "arbitrary"),
        ),
    )(features, wte_t, sums3, share, W_experts, b_all)
    return y, loss[0, 0]


def kernel(features, domain_emb, W_share, b_share, W_conv, b_conv,
           ln_gamma, ln_beta, W_gate, b_gate, W_experts, b_experts):
    return _moe(features, domain_emb, W_share, b_share, W_conv, b_conv,
                ln_gamma, ln_beta, W_gate, b_gate, W_experts, b_experts)


# speedup vs baseline: 1.1079x; 1.1079x over previous
import math

import jax
import jax.numpy as jnp
from jax import lax
from jax.experimental import pallas as pl
from jax.experimental.pallas import tpu as pltpu
from jax.experimental.pallas import tpu_sc as plsc

N = 2048
C = 1024
E = 8

_F32 = jnp.float32
_LOG8_7 = math.log(8.0) / 7.0

NW = 16
TPW = N // NW
NG = TPW // 16


AB = 512
ANB = N // AB


def _route_kernel(x_ref, de_ref, wc_ref, bc_ref, g_ref, bt_ref,
                  wg_ref, bg_ref, logits_ref):
    x = x_ref[...]
    conv = lax.dot_general(
        x, wc_ref[...], (((1,), (1,)), ((), ())),
        preferred_element_type=_F32) + bc_ref[...]
    mu = jnp.mean(conv, axis=1, keepdims=True)
    var = jnp.mean((conv - mu) ** 2, axis=1, keepdims=True)
    route = ((conv - mu) * lax.rsqrt(var + 1e-5) * g_ref[...]
             + bt_ref[...] + de_ref[...])
    logits_ref[...] = lax.dot_general(
        wg_ref[...], route, (((1,), (1,)), ((), ())),
        preferred_element_type=_F32) + bg_ref[...]


def _sc_route_body(logits_hbm, wte_hbm, sums_hbm, lv, wv, sv):
    cid = lax.axis_index("c")
    sid = lax.axis_index("s")
    wid = cid * 8 + sid

    @pl.when(sid < 8)
    def _active():
        _sc_route_worker(logits_hbm, wte_hbm, sums_hbm, lv, wv, sv, wid)


def _sc_route_worker(logits_hbm, wte_hbm, sums_hbm, lv, wv, sv, wid):
    base = wid * TPW
    pltpu.sync_copy(logits_hbm.at[:, pl.ds(base, TPW)], lv)
    msums = [jnp.zeros((16,), _F32) for _ in range(E)]
    wsums = [jnp.zeros((16,), _F32) for _ in range(E)]
    for g in range(NG):
        sl = pl.ds(g * 16, 16)
        l = [lv[e, sl] for e in range(E)]
        m = l[0]
        for e in range(1, E):
            m = jnp.maximum(m, l[e])
        ex = [jnp.exp(l[e] - m) for e in range(E)]
        z = ex[0]
        for e in range(1, E):
            z = z + ex[e]
        w = [ex[e] / z for e in range(E)]
        s = w[0] * l[0]
        for e in range(1, E):
            s = s + w[e] * l[e]
        rank = []
        for e in range(E):
            r = jnp.zeros((16,), _F32)
            for j in range(E):
                if j == e:
                    continue
                beats = w[j] > w[e]
                if j < e:
                    beats = beats | (w[j] == w[e])
                r = r + jnp.where(beats, 1.0, 0.0)
            rank.append(r)
        for e in range(E):
            thresh = jnp.exp(s - m + (rank[e] - 1.0) * _LOG8_7)
            sel = (rank[e] == 0.0) | (z > thresh)
            wv[e, sl] = jnp.where(sel, w[e], 0.0)
            msums[e] = msums[e] + jnp.where(sel, 1.0, 0.0)
            wsums[e] = wsums[e] + w[e]
    for e in range(E):
        sv[e, :] = msums[e]
        sv[E + e, :] = wsums[e]
    pltpu.sync_copy(wv, wte_hbm.at[:, pl.ds(base, TPW)])
    pltpu.sync_copy(sv, sums_hbm.at[wid])


def _mix_kernel(x_ref, wte_t_ref, sums_ref, ws_ref, we_ref, ba_ref,
                y_ref, loss_ref, wte_ref):
    s = pl.program_id(1)

    @pl.when(s == 0)
    def _init():
        ones = jnp.ones((1, N), _F32)
        wte_ref[...] = jnp.concatenate([wte_t_ref[...], ones], axis=0).T
        y_ref[...] = lax.dot_general(
            x_ref[...], ws_ref[...], (((1,), (1,)), ((), ())),
            preferred_element_type=_F32) + lax.dot_general(
            wte_ref[...], ba_ref[...], (((1,), (0,)), ((), ())),
            preferred_element_type=_F32)

    @pl.when(s > 0)
    def _expert():
        e = s - 1
        xw = lax.dot_general(
            x_ref[...], we_ref[0], (((1,), (1,)), ((), ())),
            preferred_element_type=_F32)
        onehot = (lax.broadcasted_iota(jnp.int32, (E + 1, 1), 0)
                  == e).astype(_F32)
        wcol = lax.dot_general(
            wte_ref[...], onehot, (((1,), (0,)), ((), ())),
            preferred_element_type=_F32)
        y_ref[...] += wcol * xw

    @pl.when(s == E)
    def _loss():
        sm = sums_ref[...]
        t0 = jnp.sum(sm, axis=0)
        tot = jnp.sum(t0, axis=1, keepdims=True)
        prod = tot[:E, :] * tot[E:, :] * (1.0 / (N * N))
        loss_ref[...] = jnp.sum(prod, axis=0, keepdims=True) * (
            float(E * E) / float(E))


@jax.jit
def _moe(features, domain_emb, W_share, b_share, W_conv, b_conv,
         ln_gamma, ln_beta, W_gate, b_gate, W_experts, b_experts):
    de = domain_emb.reshape(1, C)
    bc = b_conv.reshape(1, C)
    g = ln_gamma.reshape(1, C)
    bt = ln_beta.reshape(1, C)
    bg = b_gate.reshape(E, 1)
    b_all = jnp.concatenate([b_experts, b_share[None]], axis=0)

    full2 = lambda *_: (0, 0)

    logits_t = pl.pallas_call(
        _route_kernel,
        grid=(ANB,),
        in_specs=[
            pl.BlockSpec((AB, C), lambda i: (i, 0)),
            pl.BlockSpec((1, C), lambda i: (0, 0)),
            pl.BlockSpec((C, C), lambda i: (0, 0)),
            pl.BlockSpec((1, C), lambda i: (0, 0)),
            pl.BlockSpec((1, C), lambda i: (0, 0)),
            pl.BlockSpec((1, C), lambda i: (0, 0)),
            pl.BlockSpec((E, C), lambda i: (0, 0)),
            pl.BlockSpec((E, 1), lambda i: (0, 0)),
        ],
        out_specs=pl.BlockSpec((E, AB), lambda i: (0, i)),
        out_shape=jax.ShapeDtypeStruct((E, N), _F32),
    )(features, de, W_conv, bc, g, bt, W_gate, bg)

    mesh = plsc.VectorSubcoreMesh(core_axis_name="c", subcore_axis_name="s")
    wte_t, sums3 = pl.kernel(
        _sc_route_body,
        mesh=mesh,
        out_type=[
            jax.ShapeDtypeStruct((E, N), _F32),
            jax.ShapeDtypeStruct((NW, 2 * E, 16), _F32),
        ],
        scratch_types=[
            pltpu.VMEM((E, TPW), _F32),
            pltpu.VMEM((E, TPW), _F32),
            pltpu.VMEM((2 * E, 16), _F32),
        ],
    )(logits_t)

    y, loss = pl.pallas_call(
        _mix_kernel,
        grid=(1, E + 1),
        in_specs=[
            pl.BlockSpec((N, C), full2),
            pl.BlockSpec((E, N), full2),
            pl.BlockSpec((NW, 2 * E, 16), lambda i, s: (0, 0, 0)),
            pl.BlockSpec((C, C), full2),
            pl.BlockSpec((1, C, C),
                         lambda i, s: (jnp.maximum(s - 1, 0), 0, 0)),
            pl.BlockSpec((E + 1, C), full2),
        ],
        out_specs=[
            pl.BlockSpec((N, C), full2),
            pl.BlockSpec((1, 1), full2),
        ],
        out_shape=[
            jax.ShapeDtypeStruct((N, C), _F32),
            jax.ShapeDtypeStruct((1, 1), _F32),
        ],
        scratch_shapes=[
            pltpu.VMEM((N, E + 1), _F32),
        ],
        compiler_params=pltpu.CompilerParams(
            dimension_semantics=("arbitrary", "arbitrary"),
        ),
    )(features, wte_t, sums3, W_share, W_experts, b_all)
    return y, loss[0, 0]


def kernel(features, domain_emb, W_share, b_share, W_conv, b_conv,
           ln_gamma, ln_beta, W_gate, b_gate, W_experts, b_experts):
    return _moe(features, domain_emb, W_share, b_share, W_conv, b_conv,
                ln_gamma, ln_beta, W_gate, b_gate, W_experts, b_experts)
